# Initial kernel scaffold; baseline (speedup 1.0000x reference)
import jax
import jax.numpy as jnp
from jax.experimental import pallas as pl


def kernel(x, edge_index, W1, b1, W2, b2, gamma, beta):
    raise NotImplementedError("write your pallas kernel here")


# speedup vs baseline: 5.2114x; 5.2114x over previous
import functools

import jax
import jax.numpy as jnp
from jax import lax
from jax.experimental import pallas as pl
from jax.experimental.pallas import tpu as pltpu
from jax.experimental.pallas import tpu_sc as plsc

_NC = 2
_NS = 16
_LANES = 16


@functools.lru_cache(maxsize=None)
def _make_sc_agg(N, E, D):
    NW = _NC * _NS
    e_w = E // NW
    assert e_w * NW == E
    K = 80
    assert e_w % K == 0
    nchunks = e_w // K
    ZR = 128
    unit = _NS * ZR
    N_pad = ((N + unit - 1) // unit) * unit
    rows_w = N_pad // _NS
    assert D % _LANES == 0

    mesh = plsc.VectorSubcoreMesh(core_axis_name="c", subcore_axis_name="s")

    @functools.partial(
        pl.kernel,
        out_type=jax.ShapeDtypeStruct((_NC, N_pad, D), jnp.float32),
        mesh=mesh,
        scratch_types=[
            pltpu.VMEM_SHARED((N_pad, D), jnp.float32),
            pltpu.VMEM((ZR, D), jnp.float32),
            pltpu.VMEM((K,), jnp.int32),
            pltpu.VMEM((K,), jnp.int32),
            pltpu.VMEM((K, D), jnp.float32),
            pltpu.SemaphoreType.DMA,
        ],
    )
    def agg(feat, src, dst, out, acc, zbuf, src_v, dst_v, rows_v, sem):
        c = lax.axis_index("c")
        s = lax.axis_index("s")
        wid = c * _NS + s

        zero = jnp.zeros((_LANES,), jnp.float32)

        def zrow(r, carry):
            for j in range(D // _LANES):
                zbuf[r, pl.ds(j * _LANES, _LANES)] = zero
            return carry

        lax.fori_loop(0, ZR, zrow, 0)

        def zcp(t, carry):
            pltpu.sync_copy(zbuf, acc.at[pl.ds(s * rows_w + t * ZR, ZR)])
            return carry

        lax.fori_loop(0, rows_w // ZR, zcp, 0)
        plsc.subcore_barrier()

        ebase = wid * e_w

        def chunk(i, carry):
            b = pl.multiple_of(ebase + i * K, 8)
            pltpu.sync_copy(src.at[pl.ds(b, K)], src_v)
            pltpu.sync_copy(dst.at[pl.ds(b, K)], dst_v)
            pltpu.async_copy(feat.at[src_v], rows_v, sem).wait()
            pltpu.sync_copy(rows_v, acc.at[dst_v], add=True)
            return carry

        lax.fori_loop(0, nchunks, chunk, 0)
        plsc.subcore_barrier()

        pltpu.sync_copy(acc.at[pl.ds(s * rows_w, rows_w)],
                        out.at[c, pl.ds(s * rows_w, rows_w)])

    return agg


@functools.lru_cache(maxsize=None)
def _make_tc_layer(N, D, BN):
    assert N % BN == 0

    def body(acc_ref, w_ref, b_ref, o_ref):
        a = acc_ref[0] + acc_ref[1]
        o_ref[...] = jnp.tanh(
            jnp.dot(a, w_ref[...], preferred_element_type=jnp.float32)
            + b_ref[...])

    return pl.pallas_call(
        body,
        grid=(N // BN,),
        in_specs=[
            pl.BlockSpec((_NC, BN, D), lambda i: (0, i, 0)),
            pl.BlockSpec((D, D), lambda i: (0, 0)),
            pl.BlockSpec((1, D), lambda i: (0, 0)),
        ],
        out_specs=pl.BlockSpec((BN, D), lambda i: (i, 0)),
        out_shape=jax.ShapeDtypeStruct((N, D), jnp.float32),
    )


@functools.lru_cache(maxsize=None)
def _make_tc_final(N, D, BN):
    assert N % BN == 0

    def body(acc_ref, w_ref, b_ref, x_ref, g_ref, be_ref, o_ref):
        a = acc_ref[0] + acc_ref[1]
        t = jnp.tanh(
            jnp.dot(a, w_ref[...], preferred_element_type=jnp.float32)
            + b_ref[...])
        y = x_ref[...] + t
        mean = jnp.mean(y, axis=-1, keepdims=True)
        yc = y - mean
        var = jnp.mean(yc * yc, axis=-1, keepdims=True)
        o_ref[...] = yc * lax.rsqrt(var + 1e-5) * g_ref[...] + be_ref[...]

    return pl.pallas_call(
        body,
        grid=(N // BN,),
        in_specs=[
            pl.BlockSpec((_NC, BN, D), lambda i: (0, i, 0)),
            pl.BlockSpec((D, D), lambda i: (0, 0)),
            pl.BlockSpec((1, D), lambda i: (0, 0)),
            pl.BlockSpec((BN, D), lambda i: (i, 0)),
            pl.BlockSpec((1, D), lambda i: (0, 0)),
            pl.BlockSpec((1, D), lambda i: (0, 0)),
        ],
        out_specs=pl.BlockSpec((BN, D), lambda i: (i, 0)),
        out_shape=jax.ShapeDtypeStruct((N, D), jnp.float32),
    )


def kernel(x, edge_index, W1, b1, W2, b2, gamma, beta):
    N, D = x.shape
    E = edge_index.shape[1]
    BN = 1000

    src = edge_index[0]
    dst = edge_index[1]

    agg = _make_sc_agg(N, E, D)
    layer1 = _make_tc_layer(N, D, BN)
    final = _make_tc_final(N, D, BN)

    b1r = b1.reshape(1, D)
    b2r = b2.reshape(1, D)
    gr = gamma.reshape(1, D)
    br = beta.reshape(1, D)

    acc1 = agg(x, src, dst)
    h1 = layer1(acc1, W1, b1r)
    acc2 = agg(h1, src, dst)
    return final(acc2, W2, b2r, x, gr, br)
